# baseline (speedup 1.0000x reference)
import functools
import jax
import jax.numpy as jnp
from jax.experimental import pallas as pl
from jax.experimental.pallas import tpu as pltpu

B, N, INPUT_DIM = 16, 32768, 32
K_DIM, D_MODEL = 16, 1024
HIDDEN = 64
K_SEL, R_SEL, LAM = 8, 1.0, 0.5
MAX_PROXY = 16

PACK = 4
NP = N // PACK
NCHUNK = 8
PCH = NP // NCHUNK
SROWS = N // 128


def _body(x_ref, W1b_ref, b1b_ref, W2tb_ref, b2_ref, Wl_ref, bl_ref, Wp_ref,
          bp_ref, y_ref, tok_ref, s_ref):
    dn = (((1,), (1,)), ((), ()))
    for c in range(NCHUNK):
        xc = x_ref[0, pl.ds(c * PCH, PCH), :]
        h = jnp.tanh(
            jnp.dot(xc, W1b_ref[...], preferred_element_type=jnp.float32)
            + b1b_ref[...][None, :])
        e = jax.lax.dot_general(
            W2tb_ref[...], h, dn,
            preferred_element_type=jnp.float32) + b2_ref[0]
        s = jnp.maximum(e, 0.0) + jnp.log1p(jnp.exp(-jnp.abs(e)))
        s_ref[pl.ds(PACK * c, PACK), :] = s

    sal = s_ref[...]

    t = sal * (R_SEL / LAM)
    m = jnp.max(t, axis=1, keepdims=True).max(axis=0, keepdims=True)
    p = jnp.exp(t - m)
    z = jnp.sum(p, axis=1, keepdims=True).sum(axis=0, keepdims=True)
    y_ref[0] = p * (K_SEL / z)

    i0 = jax.lax.broadcasted_iota(jnp.int32, (PACK * NCHUNK, PCH), 0)
    i1 = jax.lax.broadcasted_iota(jnp.int32, (PACK * NCHUNK, PCH), 1)
    n_flat = (i0 // PACK) * (PACK * PCH) + i1 * PACK + (i0 % PACK)

    work = sal
    neg = jnp.float32(-jnp.inf)
    big = jnp.int32(2 ** 30)
    sub16 = jax.lax.broadcasted_iota(jnp.int32, (MAX_PROXY, 1), 0)
    sal_col = jnp.zeros((MAX_PROXY, 1), jnp.float32)
    idx_col = jnp.zeros((MAX_PROXY, 1), jnp.int32)
    cum_col = jnp.zeros((MAX_PROXY, 1), jnp.float32)
    for k in range(MAX_PROXY):
        mx = jnp.max(work, axis=1, keepdims=True).max(axis=0, keepdims=True)
        idx = jnp.min(jnp.where(work == mx, n_flat, big),
                      axis=1, keepdims=True).min(axis=0, keepdims=True)
        work = jnp.where(n_flat == idx, neg, work)
        cum = jnp.sum(jnp.where(n_flat <= idx, sal, 0.0),
                      axis=1, keepdims=True).sum(axis=0, keepdims=True)
        hit = sub16 == k
        sal_col = jnp.where(hit, mx, sal_col)
        idx_col = jnp.where(hit, idx, idx_col)
        cum_col = jnp.where(hit, cum, cum_col)

    pos_col = idx_col.astype(jnp.float32) * jnp.float32(1.0 / (N - 1))
    cum_col = cum_col * jnp.float32(1.0 / N)

    m_col = idx_col // PACK
    g_col = idx_col % PACK
    jm = jax.lax.broadcasted_iota(jnp.int32, (MAX_PROXY, NP), 1)
    oh_m = (jm == m_col).astype(jnp.float32)
    xp16 = jnp.dot(oh_m, x_ref[0], preferred_element_type=jnp.float32)
    li = jax.lax.broadcasted_iota(jnp.int32, (PACK * INPUT_DIM, INPUT_DIM), 0)
    fi = jax.lax.broadcasted_iota(jnp.int32, (PACK * INPUT_DIM, INPUT_DIM), 1)
    xg16 = jnp.zeros((MAX_PROXY, INPUT_DIM), jnp.float32)
    for g in range(PACK):
        sel = (li == g * INPUT_DIM + fi).astype(jnp.float32)
        msk = (g_col == g).astype(jnp.float32)
        xg16 = xg16 + jnp.dot(xp16 * msk, sel,
                              preferred_element_type=jnp.float32)

    nrm = jnp.sqrt(jnp.sum(xg16 * xg16, axis=1, keepdims=True)
                   + sal_col * sal_col + pos_col * pos_col
                   + cum_col * cum_col)
    inv = 1.0 / (nrm + 1e-6)
    Wl = Wl_ref[...]
    lift_pre = (jnp.dot(xg16, Wl[0:INPUT_DIM, :],
                        preferred_element_type=jnp.float32)
                + sal_col * Wl[INPUT_DIM:INPUT_DIM + 1, :]
                + pos_col * Wl[INPUT_DIM + 1:INPUT_DIM + 2, :]
                + cum_col * Wl[INPUT_DIM + 2:INPUT_DIM + 3, :])
    lifted = jnp.tanh(inv * lift_pre + bl_ref[...][None, :])
    tok_ref[0] = (jnp.dot(lifted, Wp_ref[...],
                          preferred_element_type=jnp.float32)
                  + bp_ref[...][None, :])


@functools.partial(jax.jit, static_argnames=("interpret",))
def _run(x, W1, b1, W2, b2, W_lift, b_lift, Wp, bp, interpret=False):
    x_p = x.reshape(B, NP, PACK * INPUT_DIM)
    zW1 = jnp.zeros((INPUT_DIM, HIDDEN), jnp.float32)
    W1b = jnp.concatenate(
        [jnp.concatenate([W1 if i == j else zW1 for j in range(PACK)], axis=1)
         for i in range(PACK)], axis=0)
    b1b = jnp.tile(b1, PACK)
    zW2 = jnp.zeros((1, HIDDEN), jnp.float32)
    W2tb = jnp.concatenate(
        [jnp.concatenate([W2.T if i == j else zW2 for j in range(PACK)],
                         axis=1) for i in range(PACK)], axis=0)

    y3, tokens = pl.pallas_call(
        _body,
        grid=(B,),
        in_specs=[
            pl.BlockSpec((1, NP, PACK * INPUT_DIM), lambda b: (b, 0, 0)),
            pl.BlockSpec((PACK * INPUT_DIM, PACK * HIDDEN), lambda b: (0, 0)),
            pl.BlockSpec((PACK * HIDDEN,), lambda b: (0,)),
            pl.BlockSpec((PACK, PACK * HIDDEN), lambda b: (0, 0)),
            pl.BlockSpec((1,), lambda b: (0,)),
            pl.BlockSpec((INPUT_DIM + 3, K_DIM), lambda b: (0, 0)),
            pl.BlockSpec((K_DIM,), lambda b: (0,)),
            pl.BlockSpec((K_DIM, D_MODEL), lambda b: (0, 0)),
            pl.BlockSpec((D_MODEL,), lambda b: (0,)),
        ],
        out_specs=[
            pl.BlockSpec((1, PACK * NCHUNK, PCH), lambda b: (b, 0, 0)),
            pl.BlockSpec((1, MAX_PROXY, D_MODEL), lambda b: (b, 0, 0)),
        ],
        out_shape=[
            jax.ShapeDtypeStruct((B, PACK * NCHUNK, PCH), jnp.float32),
            jax.ShapeDtypeStruct((B, MAX_PROXY, D_MODEL), jnp.float32),
        ],
        scratch_shapes=[pltpu.VMEM((PACK * NCHUNK, PCH), jnp.float32)],
        interpret=interpret,
    )(x_p, W1b, b1b, W2tb, b2, W_lift, b_lift, Wp, bp)

    y4 = y3.reshape(B, NCHUNK, PACK, PCH)
    phases = [y4[:, :, g, :].reshape(B, N // PACK, 1) for g in range(PACK)]
    y_star = sum(
        jnp.pad(ph, ((0, 0), (0, 0), (g, PACK - 1 - g)))
        for g, ph in enumerate(phases)).reshape(B, N)

    return tokens, y_star


def kernel(x, W1, b1, W2, b2, W_lift, b_lift, Wp, bp):
    return _run(x, W1, b1, W2, b2, W_lift, b_lift, Wp, bp)


# speedup vs baseline: 1.1901x; 1.1901x over previous
import functools
import jax
import jax.numpy as jnp
from jax.experimental import pallas as pl
from jax.experimental.pallas import tpu as pltpu

B, N, INPUT_DIM = 16, 32768, 32
K_DIM, D_MODEL = 16, 1024
HIDDEN = 64
K_SEL, R_SEL, LAM = 8, 1.0, 0.5
MAX_PROXY = 16

NCHUNK = 16
CH = N // NCHUNK
SROWS = N // 128


def _score_body(x_ref, W1t_ref, b1_ref, W2t_ref, b2_ref, y_ref, spc_ref,
                idx_ref, s_ref):
    for c in range(NCHUNK):
        xc = x_ref[0, pl.ds(c * CH, CH), :]
        ht = jnp.tanh(jax.lax.dot_general(
            W1t_ref[...], xc, (((1,), (1,)), ((), ())),
            preferred_element_type=jnp.float32)
            + b1_ref[...])
        e = jax.lax.dot_general(
            W2t_ref[...], ht, (((1,), (0,)), ((), ())),
            preferred_element_type=jnp.float32) + b2_ref[0]
        s = jnp.maximum(e, 0.0) + jnp.log1p(jnp.exp(-jnp.abs(e)))
        s_ref[pl.ds((CH // 128) * c, CH // 128), :] = s.reshape(CH // 128,
                                                               128)

    sal = s_ref[...]

    t = sal * (R_SEL / LAM)
    m = jnp.max(t, axis=1, keepdims=True).max(axis=0, keepdims=True)
    p = jnp.exp(t - m)
    z = jnp.sum(p, axis=1, keepdims=True).sum(axis=0, keepdims=True)
    y_ref[0] = p * (K_SEL / z)

    i0 = jax.lax.broadcasted_iota(jnp.int32, (SROWS, 128), 0)
    i1 = jax.lax.broadcasted_iota(jnp.int32, (SROWS, 128), 1)
    n_flat = i0 * 128 + i1

    work = sal
    neg = jnp.float32(-jnp.inf)
    big = jnp.int32(2 ** 30)
    sub16 = jax.lax.broadcasted_iota(jnp.int32, (MAX_PROXY, 1), 0)
    sal_col = jnp.zeros((MAX_PROXY, 1), jnp.float32)
    idx_col = jnp.zeros((MAX_PROXY, 1), jnp.int32)
    for k in range(MAX_PROXY):
        mx = jnp.max(work, axis=1, keepdims=True).max(axis=0, keepdims=True)
        idx = jnp.min(jnp.where(work == mx, n_flat, big),
                      axis=1, keepdims=True).min(axis=0, keepdims=True)
        work = jnp.where(n_flat == idx, neg, work)
        hit = sub16 == k
        sal_col = jnp.where(hit, mx, sal_col)
        idx_col = jnp.where(hit, idx, idx_col)

    pos_col = idx_col.astype(jnp.float32) * jnp.float32(1.0 / (N - 1))

    r_col = idx_col // 128
    l_col = idx_col % 128
    j16 = jax.lax.broadcasted_iota(jnp.int32, (MAX_PROXY, SROWS), 1)
    oh_lt = (j16 < r_col).astype(jnp.float32)
    oh_eq = (j16 == r_col).astype(jnp.float32)
    rowsums = jnp.sum(sal, axis=1, keepdims=True)
    pre = jnp.dot(oh_lt, rowsums, preferred_element_type=jnp.float32)
    rows16 = jnp.dot(oh_eq, sal, preferred_element_type=jnp.float32)
    lane128 = jax.lax.broadcasted_iota(jnp.int32, (MAX_PROXY, 128), 1)
    within = jnp.sum(jnp.where(lane128 <= l_col, rows16, 0.0),
                     axis=1, keepdims=True)
    cum_col = (pre + within) * jnp.float32(1.0 / N)

    spc_ref[0] = jnp.concatenate([sal_col, pos_col, cum_col], axis=1)
    idx_ref[0] = idx_col


def _proj_body(idx_sref, *refs):
    rows = refs[:MAX_PROXY]
    spc_ref, Wl_ref, bl_ref, Wp_ref, bp_ref, tok_ref = refs[MAX_PROXY:]
    b = pl.program_id(0)
    picked = []
    for k in range(MAX_PROXY):
        rem = idx_sref[b, k, 0] % 8
        picked.append(rows[k][0, pl.ds(rem, 1), :])
    xg16 = jnp.concatenate(picked, axis=0)
    spc = spc_ref[0]
    s16 = spc[:, 0:1]
    pos16 = spc[:, 1:2]
    cum16 = spc[:, 2:3]
    nrm = jnp.sqrt(jnp.sum(xg16 * xg16, axis=1, keepdims=True)
                   + s16 * s16 + pos16 * pos16 + cum16 * cum16)
    inv = 1.0 / (nrm + 1e-6)
    Wl = Wl_ref[...]
    lift_pre = (jnp.dot(xg16, Wl[0:INPUT_DIM, :],
                        preferred_element_type=jnp.float32)
                + s16 * Wl[INPUT_DIM:INPUT_DIM + 1, :]
                + pos16 * Wl[INPUT_DIM + 1:INPUT_DIM + 2, :]
                + cum16 * Wl[INPUT_DIM + 2:INPUT_DIM + 3, :])
    lifted = jnp.tanh(inv * lift_pre + bl_ref[...][None, :])
    tok_ref[0] = (jnp.dot(lifted, Wp_ref[...],
                          preferred_element_type=jnp.float32)
                  + bp_ref[...][None, :])


@functools.partial(jax.jit, static_argnames=("interpret",))
def _run(x, W1, b1, W2, b2, W_lift, b_lift, Wp, bp, interpret=False):
    y3, spc, idx16 = pl.pallas_call(
        _score_body,
        grid=(B,),
        in_specs=[
            pl.BlockSpec((1, N, INPUT_DIM), lambda b: (b, 0, 0)),
            pl.BlockSpec((HIDDEN, INPUT_DIM), lambda b: (0, 0)),
            pl.BlockSpec((HIDDEN, 1), lambda b: (0, 0)),
            pl.BlockSpec((1, HIDDEN), lambda b: (0, 0)),
            pl.BlockSpec((1,), lambda b: (0,)),
        ],
        out_specs=[
            pl.BlockSpec((1, SROWS, 128), lambda b: (b, 0, 0)),
            pl.BlockSpec((1, MAX_PROXY, 3), lambda b: (b, 0, 0)),
            pl.BlockSpec((1, MAX_PROXY, 1), lambda b: (b, 0, 0)),
        ],
        out_shape=[
            jax.ShapeDtypeStruct((B, SROWS, 128), jnp.float32),
            jax.ShapeDtypeStruct((B, MAX_PROXY, 3), jnp.float32),
            jax.ShapeDtypeStruct((B, MAX_PROXY, 1), jnp.int32),
        ],
        scratch_shapes=[pltpu.VMEM((SROWS, 128), jnp.float32)],
        interpret=interpret,
    )(x, W1.T, b1[:, None], W2.T, b2)

    y_star = y3.reshape(B, N)

    def row_spec(k):
        return pl.BlockSpec((1, 8, INPUT_DIM),
                            lambda b, idx: (b, idx[b, k, 0] // 8, 0))

    tokens = pl.pallas_call(
        _proj_body,
        grid_spec=pltpu.PrefetchScalarGridSpec(
            num_scalar_prefetch=1,
            grid=(B,),
            in_specs=[row_spec(k) for k in range(MAX_PROXY)] + [
                pl.BlockSpec((1, MAX_PROXY, 3), lambda b, idx: (b, 0, 0)),
                pl.BlockSpec((INPUT_DIM + 3, K_DIM), lambda b, idx: (0, 0)),
                pl.BlockSpec((K_DIM,), lambda b, idx: (0,)),
                pl.BlockSpec((K_DIM, D_MODEL), lambda b, idx: (0, 0)),
                pl.BlockSpec((D_MODEL,), lambda b, idx: (0,)),
            ],
            out_specs=pl.BlockSpec((1, MAX_PROXY, D_MODEL),
                                   lambda b, idx: (b, 0, 0)),
        ),
        out_shape=jax.ShapeDtypeStruct((B, MAX_PROXY, D_MODEL), jnp.float32),
        interpret=interpret,
    )(idx16, *([x] * MAX_PROXY), spc, W_lift, b_lift, Wp, bp)

    return tokens, y_star


def kernel(x, W1, b1, W2, b2, W_lift, b_lift, Wp, bp):
    return _run(x, W1, b1, W2, b2, W_lift, b_lift, Wp, bp)
